# baseline (speedup 1.0000x reference)
import functools

import jax
import jax.numpy as jnp
from jax import lax
from jax.experimental import pallas as pl
from jax.experimental.pallas import tpu as pltpu
from jax.experimental.pallas import tpu_sc as plsc

N = 10000
D = 128
B = 128

_info = plsc.get_sparse_core_info()
NC = _info.num_cores
NS = _info.num_subcores
NW = NC * NS

CHUNK = (N // (NW * 8)) * 8
REM = N - NW * CHUNK
assert 0 <= REM <= 128 and REM % 8 == 0

GROUPS = [16, CHUNK - 16 - 2 * 104, 104, 104]
assert sum(GROUPS) == CHUNK and all(g % 8 == 0 and 0 < g <= 128 for g in GROUPS)
OFFS = [sum(GROUPS[:i]) for i in range(len(GROUPS))]
NGROUPS = len(GROUPS)

ROWS_PER_SUB = B // NS

NROWS = (N + 127) // 128
NPAD = NROWS * 128


def _sc_segment_sum(x, batch):
    @functools.partial(
        pl.kernel,
        mesh=plsc.VectorSubcoreMesh(core_axis_name="c", subcore_axis_name="s"),
        compiler_params=pltpu.CompilerParams(use_tc_tiling_on_sc=True),
        out_type=jax.ShapeDtypeStruct((NC * B, D), jnp.float32),
        scratch_types=[
            pltpu.VMEM((CHUNK, D), jnp.float32),
            pltpu.VMEM((1, GROUPS[0]), jnp.int32),
            pltpu.VMEM((1, GROUPS[1]), jnp.int32),
            pltpu.VMEM((2, GROUPS[2]), jnp.int32),
            pltpu.VMEM((max(REM, 8), D), jnp.float32),
            pltpu.VMEM((1, max(REM, 8)), jnp.int32),
            pltpu.VMEM((ROWS_PER_SUB, D), jnp.float32),
            pltpu.VMEM_SHARED((B, D), jnp.float32),
            pltpu.SemaphoreType.DMA,
            pltpu.SemaphoreType.DMA,
            pltpu.SemaphoreType.DMA,
            pltpu.SemaphoreType.DMA,
            pltpu.SemaphoreType.DMA,
            pltpu.SemaphoreType.DMA,
        ],
    )
    def k(x_hbm, batch_hbm, out_hbm,
          xbuf, idxb0, idxb1, idxbuf, xrem, idxrem, zbuf, acc,
          sem0, sem1, sem2, sem3, semi, semz):
        c = lax.axis_index("c")
        s = lax.axis_index("s")
        wid = s * NC + c
        base = wid * CHUNK

        sems = [sem0, sem1, sem2, sem3]
        assert NGROUPS == len(sems)
        idxrefs = [idxb0.at[0], idxb1.at[0], idxbuf.at[0], idxbuf.at[1]]

        icopies = [
            pltpu.async_copy(
                batch_hbm.at[pl.ds(base + OFFS[g], GROUPS[g])],
                idxrefs[g],
                semi,
            )
            for g in range(NGROUPS)
        ]

        zvec = jnp.zeros((16,), jnp.float32)
        for r in range(ROWS_PER_SUB):
            for ch in range(D // 16):
                zbuf[r, pl.ds(ch * 16, 16)] = zvec
        row = s * ROWS_PER_SUB
        zc0 = pltpu.async_copy(zbuf, acc.at[pl.ds(row, ROWS_PER_SUB)], semz)

        xcopies = [
            pltpu.async_copy(
                x_hbm.at[pl.ds(base + OFFS[g], GROUPS[g])],
                xbuf.at[pl.ds(OFFS[g], GROUPS[g])],
                sems[g],
            )
            for g in range(NGROUPS)
        ]

        if REM:
            @pl.when(wid == 0)
            def _():
                pltpu.async_copy(
                    batch_hbm.at[pl.ds(NW * CHUNK, REM)],
                    idxrem.at[0, pl.ds(0, REM)],
                    semi,
                )
                pltpu.async_copy(
                    x_hbm.at[pl.ds(NW * CHUNK, REM)],
                    xrem.at[pl.ds(0, REM)],
                    sem1,
                )

        for cp in icopies:
            cp.wait()
        zc0.wait()
        plsc.subcore_barrier()

        scopies = []
        for g in range(NGROUPS):
            xcopies[g].wait()
            scopies.append(
                pltpu.async_copy(
                    xbuf.at[pl.ds(OFFS[g], GROUPS[g])],
                    acc.at[idxrefs[g]],
                    semz,
                    add=True,
                )
            )

        if REM:
            @pl.when(wid == 0)
            def _():
                pltpu.make_async_copy(
                    batch_hbm.at[pl.ds(NW * CHUNK, REM)],
                    idxrem.at[0, pl.ds(0, REM)],
                    semi,
                ).wait()
                pltpu.make_async_copy(
                    x_hbm.at[pl.ds(NW * CHUNK, REM)],
                    xrem.at[pl.ds(0, REM)],
                    sem1,
                ).wait()
                pltpu.sync_copy(
                    xrem.at[pl.ds(0, REM)],
                    acc.at[idxrem.at[0, pl.ds(0, REM)]],
                    add=True,
                )

        for cp in scopies:
            cp.wait()

        plsc.subcore_barrier()

        pltpu.sync_copy(
            acc.at[pl.ds(row, ROWS_PER_SUB)],
            out_hbm.at[pl.ds(c * B + row, ROWS_PER_SUB)],
        )

    return k(x, batch)


def _tc_counts(batch2d):
    def body(bat_ref, o_ref):
        seg = lax.broadcasted_iota(jnp.int32, (B, 128), 0)

        def step(r, acc):
            eq = (bat_ref[pl.ds(r, 1), :] == seg).astype(jnp.float32)
            return acc + eq

        accm = lax.fori_loop(
            0, NROWS, step, jnp.zeros((B, 128), jnp.float32)
        )
        o_ref[...] = jnp.sum(accm, axis=1, keepdims=True)

    return pl.pallas_call(
        body,
        out_shape=jax.ShapeDtypeStruct((B, 1), jnp.float32),
    )(batch2d)


def _tc_finish(partials, counts, u, w, bias):
    def body(p_ref, c_ref, u_ref, w_ref, b_ref, o_ref):
        sums = p_ref[pl.ds(0, B), :] + p_ref[pl.ds(B, B), :]
        mean = sums / jnp.maximum(c_ref[...], 1.0)
        out = lax.dot_general(
            u_ref[...], w_ref[:, pl.ds(0, D)],
            (((1,), (1,)), ((), ())), preferred_element_type=jnp.float32,
        )
        out = out + lax.dot_general(
            mean, w_ref[:, pl.ds(D, D)],
            (((1,), (1,)), ((), ())), preferred_element_type=jnp.float32,
        )
        out = out + b_ref[...]
        o_ref[...] = jnp.maximum(out, 0.0)

    return pl.pallas_call(
        body,
        out_shape=jax.ShapeDtypeStruct((B, D), jnp.float32),
    )(partials, counts, u, w, bias)


@jax.jit
def kernel(x, edge_index, edge_attr, u, batch, W, b):
    del edge_index, edge_attr
    batch = batch.astype(jnp.int32)
    partials = _sc_segment_sum(x, batch)
    batch2d = jnp.pad(batch, (0, NPAD - N), constant_values=B + 1).reshape(
        NROWS, 128
    )
    counts = _tc_counts(batch2d)
    bias = b.reshape(1, D)
    return _tc_finish(partials, counts, u, W, bias)


# speedup vs baseline: 1.0020x; 1.0020x over previous
import functools

import jax
import jax.numpy as jnp
from jax import lax
from jax.experimental import pallas as pl
from jax.experimental.pallas import tpu as pltpu
from jax.experimental.pallas import tpu_sc as plsc

N = 10000
D = 128
B = 128

try:
    _info = plsc.get_sparse_core_info()
    NC, NS = _info.num_cores, _info.num_subcores
except Exception:
    NC, NS = 2, 16
NW = NC * NS

CHUNK = (N // (NW * 8)) * 8
REM = N - NW * CHUNK
assert 0 <= REM <= 128 and REM % 8 == 0

GROUPS = [16, CHUNK - 16 - 2 * 104, 104, 104]
assert sum(GROUPS) == CHUNK and all(g % 8 == 0 and 0 < g <= 128 for g in GROUPS)
OFFS = [sum(GROUPS[:i]) for i in range(len(GROUPS))]
NGROUPS = len(GROUPS)

ROWS_PER_SUB = B // NS

NROWS = (N + 127) // 128
NPAD = NROWS * 128


def _sc_segment_sum(x, batch):
    @functools.partial(
        pl.kernel,
        mesh=plsc.VectorSubcoreMesh(core_axis_name="c", subcore_axis_name="s"),
        compiler_params=pltpu.CompilerParams(use_tc_tiling_on_sc=True),
        out_type=jax.ShapeDtypeStruct((NC * B, D), jnp.float32),
        scratch_types=[
            pltpu.VMEM((CHUNK, D), jnp.float32),
            pltpu.VMEM((1, GROUPS[0]), jnp.int32),
            pltpu.VMEM((1, GROUPS[1]), jnp.int32),
            pltpu.VMEM((2, GROUPS[2]), jnp.int32),
            pltpu.VMEM((max(REM, 8), D), jnp.float32),
            pltpu.VMEM((1, max(REM, 8)), jnp.int32),
            pltpu.VMEM((ROWS_PER_SUB, D), jnp.float32),
            pltpu.VMEM_SHARED((B, D), jnp.float32),
            pltpu.SemaphoreType.DMA,
            pltpu.SemaphoreType.DMA,
            pltpu.SemaphoreType.DMA,
            pltpu.SemaphoreType.DMA,
            pltpu.SemaphoreType.DMA,
            pltpu.SemaphoreType.DMA,
        ],
    )
    def k(x_hbm, batch_hbm, out_hbm,
          xbuf, idxb0, idxb1, idxbuf, xrem, idxrem, zbuf, acc,
          sem0, sem1, sem2, sem3, semi, semz):
        c = lax.axis_index("c")
        s = lax.axis_index("s")
        wid = s * NC + c
        base = wid * CHUNK

        sems = [sem0, sem1, sem2, sem3]
        assert NGROUPS == len(sems)
        idxrefs = [idxb0.at[0], idxb1.at[0], idxbuf.at[0], idxbuf.at[1]]

        icopies = [
            pltpu.async_copy(
                batch_hbm.at[pl.ds(base + OFFS[g], GROUPS[g])],
                idxrefs[g],
                semi,
            )
            for g in range(NGROUPS)
        ]

        zvec = jnp.zeros((16,), jnp.float32)
        for r in range(ROWS_PER_SUB):
            for ch in range(D // 16):
                zbuf[r, pl.ds(ch * 16, 16)] = zvec
        row = s * ROWS_PER_SUB
        zc0 = pltpu.async_copy(zbuf, acc.at[pl.ds(row, ROWS_PER_SUB)], semz)

        xcopies = [
            pltpu.async_copy(
                x_hbm.at[pl.ds(base + OFFS[g], GROUPS[g])],
                xbuf.at[pl.ds(OFFS[g], GROUPS[g])],
                sems[g],
            )
            for g in range(NGROUPS)
        ]

        if REM:
            @pl.when(wid == 0)
            def _():
                pltpu.async_copy(
                    batch_hbm.at[pl.ds(NW * CHUNK, REM)],
                    idxrem.at[0, pl.ds(0, REM)],
                    semi,
                )
                pltpu.async_copy(
                    x_hbm.at[pl.ds(NW * CHUNK, REM)],
                    xrem.at[pl.ds(0, REM)],
                    sem1,
                )

        for cp in icopies:
            cp.wait()
        zc0.wait()
        plsc.subcore_barrier()

        scopies = []
        for g in range(NGROUPS):
            xcopies[g].wait()
            scopies.append(
                pltpu.async_copy(
                    xbuf.at[pl.ds(OFFS[g], GROUPS[g])],
                    acc.at[idxrefs[g]],
                    semz,
                    add=True,
                )
            )

        if REM:
            @pl.when(wid == 0)
            def _():
                pltpu.make_async_copy(
                    batch_hbm.at[pl.ds(NW * CHUNK, REM)],
                    idxrem.at[0, pl.ds(0, REM)],
                    semi,
                ).wait()
                pltpu.make_async_copy(
                    x_hbm.at[pl.ds(NW * CHUNK, REM)],
                    xrem.at[pl.ds(0, REM)],
                    sem1,
                ).wait()
                pltpu.sync_copy(
                    xrem.at[pl.ds(0, REM)],
                    acc.at[idxrem.at[0, pl.ds(0, REM)]],
                    add=True,
                )

        for cp in scopies:
            cp.wait()

        plsc.subcore_barrier()

        pltpu.sync_copy(
            acc.at[pl.ds(row, ROWS_PER_SUB)],
            out_hbm.at[pl.ds(c * B + row, ROWS_PER_SUB)],
        )

    return k(x, batch)


def _tc_counts(batch2d):
    def body(bat_ref, o_ref):
        seg = lax.broadcasted_iota(jnp.int32, (B, 128), 0)

        def step(r, acc):
            eq = (bat_ref[pl.ds(r, 1), :] == seg).astype(jnp.float32)
            return acc + eq

        accm = lax.fori_loop(
            0, NROWS, step, jnp.zeros((B, 128), jnp.float32)
        )
        o_ref[...] = jnp.sum(accm, axis=1, keepdims=True)

    return pl.pallas_call(
        body,
        out_shape=jax.ShapeDtypeStruct((B, 1), jnp.float32),
    )(batch2d)


def _tc_finish(partials, counts, u, w, bias):
    def body(p_ref, c_ref, u_ref, w_ref, b_ref, o_ref):
        sums = p_ref[pl.ds(0, B), :] + p_ref[pl.ds(B, B), :]
        mean = sums / jnp.maximum(c_ref[...], 1.0)
        out = lax.dot_general(
            u_ref[...], w_ref[:, pl.ds(0, D)],
            (((1,), (1,)), ((), ())), preferred_element_type=jnp.float32,
        )
        out = out + lax.dot_general(
            mean, w_ref[:, pl.ds(D, D)],
            (((1,), (1,)), ((), ())), preferred_element_type=jnp.float32,
        )
        out = out + b_ref[...]
        o_ref[...] = jnp.maximum(out, 0.0)

    return pl.pallas_call(
        body,
        out_shape=jax.ShapeDtypeStruct((B, D), jnp.float32),
    )(partials, counts, u, w, bias)


@jax.jit
def kernel(x, edge_index, edge_attr, u, batch, W, b):
    del edge_index, edge_attr
    batch = batch.astype(jnp.int32)
    partials = _sc_segment_sum(x, batch)
    batch2d = jnp.pad(batch, (0, NPAD - N), constant_values=B + 1).reshape(
        NROWS, 128
    )
    counts = _tc_counts(batch2d)
    bias = b.reshape(1, D)
    return _tc_finish(partials, counts, u, W, bias)
